# initial kernel scaffold (unmeasured)
import jax
import jax.numpy as jnp
from jax import lax
from jax.experimental import pallas as pl
from jax.experimental.pallas import tpu as pltpu


def kernel(x, dest):
    m, n = x.shape

    my_z = lax.axis_index("z")

    send_flag = (dest != my_z).astype(jnp.int32)
    perm = jnp.argsort(send_flag, stable=True)
    xs = x[perm].astype(jnp.bfloat16)
    n_keep = jnp.sum(1 - send_flag).astype(jnp.int32)
    xs_pad = jnp.concatenate([xs, jnp.zeros_like(xs)], axis=0)
    send_buf = lax.dynamic_slice(xs_pad, (n_keep, 0), (m, n))

    def body(nk_ref, xs_ref, send_ref, out_ref, recv_ref, asm_ref,
             send_sem, recv_sem):
        zi = lax.axis_index("z")
        xi = lax.axis_index("x")
        yi = lax.axis_index("y")
        peer = (xi, yi, 1 - zi)

        barrier_sem = pltpu.get_barrier_semaphore()
        pl.semaphore_signal(
            barrier_sem, inc=1, device_id=peer,
            device_id_type=pl.DeviceIdType.MESH,
        )
        pl.semaphore_wait(barrier_sem, 1)

        rdma = pltpu.make_async_remote_copy(
            src_ref=send_ref,
            dst_ref=recv_ref,
            send_sem=send_sem,
            recv_sem=recv_sem,
            device_id=peer,
            device_id_type=pl.DeviceIdType.MESH,
        )
        rdma.start()

        asm_ref[pl.ds(0, m), :] = xs_ref[:, :]

        rdma.wait()

        nk = nk_ref[0]
        asm_ref[pl.ds(nk, m), :] = recv_ref[:, :]
        out_ref[:, :] = asm_ref[pl.ds(0, m), :]

    out = pl.pallas_call(
        body,
        out_shape=jax.ShapeDtypeStruct((m, n), jnp.bfloat16),
        in_specs=[
            pl.BlockSpec(memory_space=pltpu.SMEM),
            pl.BlockSpec(memory_space=pltpu.VMEM),
            pl.BlockSpec(memory_space=pltpu.VMEM),
        ],
        out_specs=pl.BlockSpec(memory_space=pltpu.VMEM),
        scratch_shapes=[
            pltpu.VMEM((m, n), jnp.bfloat16),
            pltpu.VMEM((2 * m, n), jnp.bfloat16),
            pltpu.SemaphoreType.DMA,
            pltpu.SemaphoreType.DMA,
        ],
        compiler_params=pltpu.CompilerParams(collective_id=0),
    )(n_keep.reshape(1), xs, send_buf)
    return out


# baseline (device time: 89072 ns/iter reference)
import jax
import jax.numpy as jnp
from jax import lax
from jax.experimental import pallas as pl
from jax.experimental.pallas import tpu as pltpu


def kernel(x, dest):
    m, n = x.shape

    my_z = lax.axis_index("z")

    send_flag = (dest != my_z).astype(jnp.int32)
    perm = jnp.argsort(send_flag, stable=True)
    xs = x[perm].astype(jnp.bfloat16)
    n_keep = jnp.sum(1 - send_flag).astype(jnp.int32)
    xs_pad = jnp.concatenate([xs, jnp.zeros_like(xs)], axis=0)
    send_buf = lax.dynamic_slice(xs_pad, (n_keep, 0), (m, n))

    def body(send_ref, recv_ref, send_sem, recv_sem):
        zi = lax.axis_index("z")
        xi = lax.axis_index("x")
        yi = lax.axis_index("y")
        peer = (xi, yi, 1 - zi)

        barrier_sem = pltpu.get_barrier_semaphore()
        pl.semaphore_signal(
            barrier_sem, inc=1, device_id=peer,
            device_id_type=pl.DeviceIdType.MESH,
        )
        pl.semaphore_wait(barrier_sem, 1)

        rdma = pltpu.make_async_remote_copy(
            src_ref=send_ref,
            dst_ref=recv_ref,
            send_sem=send_sem,
            recv_sem=recv_sem,
            device_id=peer,
            device_id_type=pl.DeviceIdType.MESH,
        )
        rdma.start()
        rdma.wait()

    recv = pl.pallas_call(
        body,
        out_shape=jax.ShapeDtypeStruct((m, n), jnp.bfloat16),
        in_specs=[pl.BlockSpec(memory_space=pltpu.VMEM)],
        out_specs=pl.BlockSpec(memory_space=pltpu.VMEM),
        scratch_shapes=[
            pltpu.SemaphoreType.DMA,
            pltpu.SemaphoreType.DMA,
        ],
        compiler_params=pltpu.CompilerParams(collective_id=0),
    )(send_buf)

    n_recv = m - n_keep
    recv_pad = jnp.concatenate([recv, jnp.zeros_like(recv)], axis=0)
    base, upd, off = lax.cond(
        my_z == 0,
        lambda: (xs_pad, recv, n_keep),
        lambda: (recv_pad, xs, n_recv),
    )
    assembled = lax.dynamic_update_slice(base, upd, (off, 0))
    return assembled[:m]


# device time: 51502 ns/iter; 1.7295x vs baseline; 1.7295x over previous
import jax
import jax.numpy as jnp
from jax import lax
from jax.experimental import pallas as pl
from jax.experimental.pallas import tpu as pltpu

CH = 128


def kernel(x, dest):
    m, n = x.shape
    max_chunks = m // CH

    my_z = lax.axis_index("z")

    perm = jnp.argsort(dest, stable=True)
    xs = x[perm].astype(jnp.bfloat16)
    c0 = jnp.sum(dest == 0).astype(jnp.int32)

    k_rows = jnp.where(my_z == 0, m - c0, c0).astype(jnp.int32)
    send_off = jnp.where(my_z == 0, c0, 0).astype(jnp.int32)
    xs_pad = jnp.concatenate([xs, jnp.zeros_like(xs)], axis=0)
    send_buf = lax.dynamic_slice(xs_pad, (send_off, 0), (m, n))
    scalars = jnp.stack([c0, k_rows]).astype(jnp.int32)

    def body(sc_ref, xs_ref, send_ref, out_ref, recv_ref,
             send_sems, recv_sems):
        zi = lax.axis_index("z")
        xi = lax.axis_index("x")
        yi = lax.axis_index("y")
        peer = (xi, yi, 1 - zi)
        c0s = sc_ref[0]
        ks = sc_ref[1]
        n_chunks = (ks + CH - 1) // CH

        barrier_sem = pltpu.get_barrier_semaphore()
        pl.semaphore_signal(
            barrier_sem, inc=1, device_id=peer,
            device_id_type=pl.DeviceIdType.MESH,
        )
        pl.semaphore_wait(barrier_sem, 1)

        def chunk_rdma(i):
            o = pl.multiple_of(i * CH, CH)
            return pltpu.make_async_remote_copy(
                src_ref=send_ref.at[pl.ds(o, CH)],
                dst_ref=recv_ref.at[pl.ds(o, CH)],
                send_sem=send_sems.at[i],
                recv_sem=recv_sems.at[i],
                device_id=peer,
                device_id_type=pl.DeviceIdType.MESH,
            )

        def send_body(i, _):
            chunk_rdma(i).start()
            return 0

        lax.fori_loop(0, n_chunks, send_body, 0)

        def recv_body(i, _):
            chunk_rdma(i).wait_recv()
            return 0

        lax.fori_loop(0, n_chunks, recv_body, 0)

        shift = jnp.where(zi == 0, c0s, 0)
        rolled = pltpu.roll(recv_ref[:, :], shift, 0)
        rowid = lax.broadcasted_iota(jnp.int32, (m, n), 0)
        take_xs = (rowid < c0s) == (zi == 0)
        out_ref[:, :] = jnp.where(take_xs, xs_ref[:, :], rolled)

        def wait_send_body(i, _):
            chunk_rdma(i).wait_send()
            return 0

        lax.fori_loop(0, n_chunks, wait_send_body, 0)

    return pl.pallas_call(
        body,
        out_shape=jax.ShapeDtypeStruct((m, n), jnp.bfloat16),
        in_specs=[
            pl.BlockSpec(memory_space=pltpu.SMEM),
            pl.BlockSpec(memory_space=pltpu.VMEM),
            pl.BlockSpec(memory_space=pltpu.VMEM),
        ],
        out_specs=pl.BlockSpec(memory_space=pltpu.VMEM),
        scratch_shapes=[
            pltpu.VMEM((m, n), jnp.bfloat16),
            pltpu.SemaphoreType.DMA((max_chunks,)),
            pltpu.SemaphoreType.DMA((max_chunks,)),
        ],
        compiler_params=pltpu.CompilerParams(collective_id=0),
    )(scalars, xs, send_buf)
